# baseline (speedup 1.0000x reference)
import functools

import jax
import jax.numpy as jnp
from jax import lax
from jax.experimental import pallas as pl
from jax.experimental.pallas import tpu as pltpu
from jax.experimental.pallas import tpu_sc as plsc

EMBED = 64
NBUF = 2
LANE = 16


def _make_transpose_kernel(v: int):
    info = plsc.get_sparse_core_info()
    nc, ns = info.num_cores, info.num_subcores
    nw = nc * ns
    nt = v // 128
    per_w = nt // nw
    n_extra = nt - per_w * nw
    assert per_w % NBUF == 0

    mesh = plsc.VectorSubcoreMesh(core_axis_name="c", subcore_axis_name="s")

    @functools.partial(
        pl.kernel,
        out_type=jax.ShapeDtypeStruct((v * EMBED // 128, 128), jnp.float32),
        mesh=mesh,
        compiler_params=pltpu.CompilerParams(needs_layout_passes=False),
        scratch_types=(
            [pltpu.VMEM((EMBED, 128), jnp.float32)] * NBUF
            + [pltpu.VMEM((EMBED, 128), jnp.float32)] * NBUF
            + [pltpu.SemaphoreType.DMA] * (2 * NBUF)),
    )
    def tr_kernel(wt_hbm, wtail_hbm, out_hbm,
                  ibuf0, ibuf1, obuf0, obuf1, si0, si1, so0, so1):
        ibufs, obufs = (ibuf0, ibuf1), (obuf0, obuf1)
        si, so = (si0, si1), (so0, so1)
        wid = lax.axis_index("s") * nc + lax.axis_index("c")
        base = wid * per_w

        iota16 = lax.iota(jnp.int32, 16)
        jvecs = [j0 * LANE + iota16 for j0 in range(128 // LANE)]
        rvecs = [jv >> 1 for jv in jvecs]
        cbase = [(jv & 1) * EMBED for jv in jvecs]

        def in_start(tc, b):
            pltpu.async_copy(
                wt_hbm.at[:, pl.ds(tc * 128, 128)], ibufs[b], si[b])

        def in_wait(b):
            pltpu.make_async_copy(
                wt_hbm.at[:, pl.ds(0, 128)], ibufs[b], si[b]).wait()

        def out_start(tc, b):
            pltpu.async_copy(
                obufs[b], out_hbm.at[pl.ds(tc * EMBED, EMBED)], so[b])

        def out_wait(b):
            pltpu.make_async_copy(
                obufs[b], out_hbm.at[pl.ds(0, EMBED)], so[b]).wait()

        def transpose(b):
            @plsc.parallel_loop(0, LANE, unroll=2)
            def phase_body(p):
                rotv = (p + iota16) & 15
                for e0 in range(EMBED // LANE):
                    evec = e0 * LANE + rotv
                    for j0 in range(128 // LANE):
                        wv = plsc.load_gather(ibufs[b], [evec, jvecs[j0]])
                        plsc.store_scatter(
                            obufs[b], [rvecs[j0], cbase[j0] + evec], wv)

        for b in range(NBUF):
            in_start(base + b, b)
        for b in range(NBUF):
            in_wait(b)
            transpose(b)
            out_start(base + b, b)
            in_start(base + b + NBUF, b)

        def body(g, carry):
            for b in range(NBUF):
                t = g * NBUF + b
                in_wait(b)
                out_wait(b)
                transpose(b)
                out_start(base + t, b)
                in_start(base + t + NBUF, b)
            return carry
        lax.fori_loop(1, per_w // NBUF - 1, body, 0)

        for b in range(NBUF):
            t = per_w - NBUF + b
            in_wait(b)
            out_wait(b)
            transpose(b)
            out_start(base + t, b)
        for b in range(NBUF):
            out_wait(b)

        @pl.when(wid < n_extra)
        def _():
            tc = per_w * nw + wid
            in_start(tc, 0)
            in_wait(0)
            transpose(0)
            out_start(tc, 0)
            out_wait(0)

        @pl.when(wid == n_extra)
        def _():
            pltpu.sync_copy(wtail_hbm, ibufs[0].at[pl.ds(0, 32)])
            pltpu.sync_copy(ibufs[0].at[pl.ds(0, 32)],
                            out_hbm.at[pl.ds(v * EMBED // 128 - 32, 32)])

    return tr_kernel


def _make_emb_kernel(n_b: int, n_l: int, n_pos: int):
    info = plsc.get_sparse_core_info()
    nc, ns = info.num_cores, info.num_subcores
    nw = nc * ns
    assert n_b == nw * 128 and n_l % NBUF == 0

    mesh = plsc.VectorSubcoreMesh(core_axis_name="c", subcore_axis_name="s")

    @functools.partial(
        pl.kernel,
        out_type=jax.ShapeDtypeStruct((n_l, EMBED // 8, nw, 1024),
                                      jnp.float32),
        mesh=mesh,
        compiler_params=pltpu.CompilerParams(
            use_tc_tiling_on_sc=False, needs_layout_passes=False),
        scratch_types=[
            pltpu.VMEM((n_l, 128), jnp.int32),
            pltpu.VMEM((n_l, 128), jnp.int32),
            pltpu.VMEM((n_pos, EMBED), jnp.float32),
        ] + [pltpu.VMEM((128, EMBED), jnp.float32)] * NBUF
          + [pltpu.VMEM((EMBED // 8, 1024), jnp.float32)] * NBUF
          + [pltpu.SemaphoreType.DMA] * (2 * NBUF),
    )
    def emb_kernel(ids_hbm, pids_hbm, wtab_hbm, ptab_hbm, out_hbm,
                   idx_v, pidx_v, ptab_v, wbuf0, wbuf1, obuf0, obuf1,
                   sg0, sg1, ss0, ss1):
        wbufs = (wbuf0, wbuf1)
        obufs = (obuf0, obuf1)
        sg, ss = (sg0, sg1), (ss0, ss1)
        wid = lax.axis_index("s") * nc + lax.axis_index("c")
        pltpu.sync_copy(ptab_hbm, ptab_v)
        pltpu.sync_copy(ids_hbm.at[:, pl.ds(wid * 128, 128)], idx_v)
        pltpu.sync_copy(pids_hbm.at[:, pl.ds(wid * 128, 128)], pidx_v)

        iota16 = lax.iota(jnp.int32, 16)
        lane_vecs = [lg * LANE + iota16 for lg in range(128 // LANE)]

        def gather_start(l, b):
            pltpu.async_copy(
                wtab_hbm.at[idx_v.at[l, pl.ds(0, 64)]],
                wbufs[b].at[pl.ds(0, 64)], sg[b])
            pltpu.async_copy(
                wtab_hbm.at[idx_v.at[l, pl.ds(64, 64)]],
                wbufs[b].at[pl.ds(64, 64)], sg[b])

        def gather_wait(b):
            for h in range(2):
                pltpu.make_async_copy(
                    wtab_hbm.at[idx_v.at[0, pl.ds(0, 64)]],
                    wbufs[b].at[pl.ds(h * 64, 64)], sg[b]).wait()

        def add(l, b):
            pvecs = [pidx_v[l, pl.ds(lg * LANE, LANE)]
                     for lg in range(128 // LANE)]

            @plsc.parallel_loop(0, LANE, unroll=2)
            def rot_body(rot):
                rotv = (rot + iota16) & 15
                for e_blk in range(EMBED // LANE):
                    crot = e_blk * LANE + rotv
                    trv = crot >> 3
                    inner = (crot & 7) * 128
                    for lg in range(128 // LANE):
                        wv = plsc.load_gather(
                            wbufs[b], [lane_vecs[lg], crot])
                        pv = plsc.load_gather(ptab_v, [pvecs[lg], crot])
                        plsc.store_scatter(
                            obufs[b], [trv, inner + lane_vecs[lg]], wv + pv)

        def scatter_start(l, b):
            pltpu.async_copy(obufs[b], out_hbm.at[l, :, wid], ss[b])

        def scatter_wait(b):
            pltpu.make_async_copy(obufs[b], out_hbm.at[0, :, wid], ss[b]).wait()

        for b in range(NBUF):
            gather_start(b, b)
        for b in range(NBUF):
            gather_wait(b)
            add(b, b)
            scatter_start(b, b)
            gather_start(b + NBUF, b)

        def body(g, carry):
            for b in range(NBUF):
                l = g * NBUF + b
                gather_wait(b)
                scatter_wait(b)
                add(l, b)
                scatter_start(l, b)
                gather_start(l + NBUF, b)
            return carry
        lax.fori_loop(1, n_l // NBUF - 1, body, 0)

        for b in range(NBUF):
            l = n_l - NBUF + b
            gather_wait(b)
            scatter_wait(b)
            add(l, b)
            scatter_start(l, b)
        for b in range(NBUF):
            scatter_wait(b)

    return emb_kernel


def kernel(input_ids, position_ids, word_embeddings, position_embeddings):
    b, l = input_ids.shape
    n_pos = position_embeddings.shape[0]
    v = word_embeddings.shape[0]
    tr = _make_transpose_kernel(v)
    wtail = word_embeddings[v - (v % 128):].reshape(-1, 128)
    wlin = tr(word_embeddings.T, wtail).reshape(v, EMBED)
    emb = _make_emb_kernel(b, l, n_pos)
    o = emb(input_ids.T, position_ids.T, wlin, position_embeddings)
    return (o.reshape(l, EMBED // 8, b // 128, 8, 128)
            .transpose(2, 4, 0, 1, 3).reshape(b, l, EMBED))


# speedup vs baseline: 1.0171x; 1.0171x over previous
import functools

import jax
import jax.numpy as jnp
from jax import lax
from jax.experimental import pallas as pl
from jax.experimental.pallas import tpu as pltpu
from jax.experimental.pallas import tpu_sc as plsc

EMBED = 64
NBUF = 2
LANE = 16


def _make_transpose_kernel(v: int):
    info = plsc.get_sparse_core_info()
    nc, ns = info.num_cores, info.num_subcores
    nw = nc * ns
    nt = v // 128
    per_w = nt // nw
    n_extra = nt - per_w * nw
    assert per_w % NBUF == 0

    mesh = plsc.VectorSubcoreMesh(core_axis_name="c", subcore_axis_name="s")

    @functools.partial(
        pl.kernel,
        out_type=jax.ShapeDtypeStruct((v * EMBED // 128, 128), jnp.float32),
        mesh=mesh,
        compiler_params=pltpu.CompilerParams(needs_layout_passes=False),
        scratch_types=(
            [pltpu.VMEM((EMBED, 128), jnp.float32)] * NBUF
            + [pltpu.VMEM((EMBED, 128), jnp.float32)] * NBUF
            + [pltpu.SemaphoreType.DMA] * (2 * NBUF)),
    )
    def tr_kernel(wt_hbm, wtail_hbm, out_hbm,
                  ibuf0, ibuf1, obuf0, obuf1, si0, si1, so0, so1):
        ibufs, obufs = (ibuf0, ibuf1), (obuf0, obuf1)
        si, so = (si0, si1), (so0, so1)
        wid = lax.axis_index("s") * nc + lax.axis_index("c")
        base = wid * per_w

        iota16 = lax.iota(jnp.int32, 16)
        jvecs = [j0 * LANE + iota16 for j0 in range(128 // LANE)]
        rvecs = [jv >> 1 for jv in jvecs]
        cbase = [(jv & 1) * EMBED for jv in jvecs]

        def in_start(tc, b):
            pltpu.async_copy(
                wt_hbm.at[:, pl.ds(tc * 128, 128)], ibufs[b], si[b])

        def in_wait(b):
            pltpu.make_async_copy(
                wt_hbm.at[:, pl.ds(0, 128)], ibufs[b], si[b]).wait()

        def out_start(tc, b):
            pltpu.async_copy(
                obufs[b], out_hbm.at[pl.ds(tc * EMBED, EMBED)], so[b])

        def out_wait(b):
            pltpu.make_async_copy(
                obufs[b], out_hbm.at[pl.ds(0, EMBED)], so[b]).wait()

        def transpose(b):
            @plsc.parallel_loop(0, LANE)
            def phase_body(p):
                rotv = (p + iota16) & 15
                for e0 in range(EMBED // LANE):
                    evec = e0 * LANE + rotv
                    for j0 in range(128 // LANE):
                        wv = plsc.load_gather(ibufs[b], [evec, jvecs[j0]])
                        plsc.store_scatter(
                            obufs[b], [rvecs[j0], cbase[j0] + evec], wv)

        for b in range(NBUF):
            in_start(base + b, b)
        for b in range(NBUF):
            in_wait(b)
            transpose(b)
            out_start(base + b, b)
            in_start(base + b + NBUF, b)

        def body(g, carry):
            for b in range(NBUF):
                t = g * NBUF + b
                in_wait(b)
                out_wait(b)
                transpose(b)
                out_start(base + t, b)
                in_start(base + t + NBUF, b)
            return carry
        lax.fori_loop(1, per_w // NBUF - 1, body, 0)

        for b in range(NBUF):
            t = per_w - NBUF + b
            in_wait(b)
            out_wait(b)
            transpose(b)
            out_start(base + t, b)
        for b in range(NBUF):
            out_wait(b)

        @pl.when(wid < n_extra)
        def _():
            tc = per_w * nw + wid
            in_start(tc, 0)
            in_wait(0)
            transpose(0)
            out_start(tc, 0)
            out_wait(0)

        @pl.when(wid == n_extra)
        def _():
            pltpu.sync_copy(wtail_hbm, ibufs[0].at[pl.ds(0, 32)])
            pltpu.sync_copy(ibufs[0].at[pl.ds(0, 32)],
                            out_hbm.at[pl.ds(v * EMBED // 128 - 32, 32)])

    return tr_kernel


def _make_emb_kernel(n_b: int, n_l: int, n_pos: int):
    info = plsc.get_sparse_core_info()
    nc, ns = info.num_cores, info.num_subcores
    nw = nc * ns
    assert n_b == nw * 128 and n_l % NBUF == 0

    mesh = plsc.VectorSubcoreMesh(core_axis_name="c", subcore_axis_name="s")

    @functools.partial(
        pl.kernel,
        out_type=jax.ShapeDtypeStruct((n_l, EMBED // 8, nw, 1024),
                                      jnp.float32),
        mesh=mesh,
        compiler_params=pltpu.CompilerParams(
            use_tc_tiling_on_sc=False, needs_layout_passes=False),
        scratch_types=[
            pltpu.VMEM((n_l, 128), jnp.int32),
            pltpu.VMEM((n_l, 128), jnp.int32),
            pltpu.VMEM((n_pos, EMBED), jnp.float32),
        ] + [pltpu.VMEM((128, EMBED), jnp.float32)] * NBUF
          + [pltpu.VMEM((EMBED // 8, 1024), jnp.float32)] * NBUF
          + [pltpu.SemaphoreType.DMA] * (2 * NBUF),
    )
    def emb_kernel(ids_hbm, pids_hbm, wtab_hbm, ptab_hbm, out_hbm,
                   idx_v, pidx_v, ptab_v, wbuf0, wbuf1, obuf0, obuf1,
                   sg0, sg1, ss0, ss1):
        wbufs = (wbuf0, wbuf1)
        obufs = (obuf0, obuf1)
        sg, ss = (sg0, sg1), (ss0, ss1)
        wid = lax.axis_index("s") * nc + lax.axis_index("c")
        pltpu.sync_copy(ptab_hbm, ptab_v)
        pltpu.sync_copy(ids_hbm.at[:, pl.ds(wid * 128, 128)], idx_v)
        pltpu.sync_copy(pids_hbm.at[:, pl.ds(wid * 128, 128)], pidx_v)

        iota16 = lax.iota(jnp.int32, 16)
        lane_vecs = [lg * LANE + iota16 for lg in range(128 // LANE)]

        def gather_start(l, b):
            pltpu.async_copy(
                wtab_hbm.at[idx_v.at[l, pl.ds(0, 64)]],
                wbufs[b].at[pl.ds(0, 64)], sg[b])
            pltpu.async_copy(
                wtab_hbm.at[idx_v.at[l, pl.ds(64, 64)]],
                wbufs[b].at[pl.ds(64, 64)], sg[b])

        def gather_wait(b):
            for h in range(2):
                pltpu.make_async_copy(
                    wtab_hbm.at[idx_v.at[0, pl.ds(0, 64)]],
                    wbufs[b].at[pl.ds(h * 64, 64)], sg[b]).wait()

        def add(l, b):
            pvecs = [pidx_v[l, pl.ds(lg * LANE, LANE)]
                     for lg in range(128 // LANE)]

            @plsc.parallel_loop(0, LANE)
            def rot_body(rot):
                rotv = (rot + iota16) & 15
                for e_blk in range(EMBED // LANE):
                    crot = e_blk * LANE + rotv
                    trv = crot >> 3
                    inner = (crot & 7) * 128
                    for lg in range(128 // LANE):
                        wv = plsc.load_gather(
                            wbufs[b], [lane_vecs[lg], crot])
                        pv = plsc.load_gather(ptab_v, [pvecs[lg], crot])
                        plsc.store_scatter(
                            obufs[b], [trv, inner + lane_vecs[lg]], wv + pv)

        def scatter_start(l, b):
            pltpu.async_copy(obufs[b], out_hbm.at[l, :, wid], ss[b])

        def scatter_wait(b):
            pltpu.make_async_copy(obufs[b], out_hbm.at[0, :, wid], ss[b]).wait()

        for b in range(NBUF):
            gather_start(b, b)
        for b in range(NBUF):
            gather_wait(b)
            add(b, b)
            scatter_start(b, b)
            gather_start(b + NBUF, b)

        def body(g, carry):
            for b in range(NBUF):
                l = g * NBUF + b
                gather_wait(b)
                scatter_wait(b)
                add(l, b)
                scatter_start(l, b)
                gather_start(l + NBUF, b)
            return carry
        lax.fori_loop(1, n_l // NBUF - 1, body, 0)

        for b in range(NBUF):
            l = n_l - NBUF + b
            gather_wait(b)
            scatter_wait(b)
            add(l, b)
            scatter_start(l, b)
        for b in range(NBUF):
            scatter_wait(b)

    return emb_kernel


def kernel(input_ids, position_ids, word_embeddings, position_embeddings):
    b, l = input_ids.shape
    n_pos = position_embeddings.shape[0]
    v = word_embeddings.shape[0]
    tr = _make_transpose_kernel(v)
    wtail = word_embeddings[v - (v % 128):].reshape(-1, 128)
    wlin = tr(word_embeddings.T, wtail).reshape(v, EMBED)
    emb = _make_emb_kernel(b, l, n_pos)
    o = emb(input_ids.T, position_ids.T, wlin, position_embeddings)
    return (o.reshape(l, EMBED // 8, b // 128, 8, 128)
            .transpose(2, 4, 0, 1, 3).reshape(b, l, EMBED))
